# Initial kernel scaffold; baseline (speedup 1.0000x reference)
import functools

import jax
import jax.numpy as jnp
from jax import lax
from jax.experimental import pallas as pl
from jax.experimental.pallas import tpu as pltpu
from jax.experimental.pallas import tpu_sc as plsc

NC = 2
NS = 16
NW = NC * NS
GW = 128


def _tc_head(x, W, a_src, a_dst):
  N, _ = x.shape
  H = W.shape[1]

  def body(x_ref, w_ref, s_ref, d_ref, h_ref, hs_ref, hd_ref):
    h = jnp.dot(x_ref[...], w_ref[...], preferred_element_type=jnp.float32)
    h_ref[...] = h
    hs_ref[...] = jnp.dot(h, s_ref[...], preferred_element_type=jnp.float32)
    hd_ref[...] = jnp.dot(h, d_ref[...], preferred_element_type=jnp.float32)

  return pl.pallas_call(
      body,
      out_shape=[
          jax.ShapeDtypeStruct((N, H), jnp.float32),
          jax.ShapeDtypeStruct((N, 1), jnp.float32),
          jax.ShapeDtypeStruct((N, 1), jnp.float32),
      ],
  )(x, W, a_src.reshape(H, 1), a_dst.reshape(H, 1))


def _tc_mid(pr, b, g, be, W, a_src, a_dst):
  _, N, H = pr.shape
  H2 = W.shape[1]

  def body(pr_ref, b_ref, g_ref, be_ref, w_ref, s_ref, d_ref,
           h_ref, hs_ref, hd_ref):
    gat = pr_ref[0] + pr_ref[1] + b_ref[...]
    mu = jnp.mean(gat, axis=0, keepdims=True)
    var = jnp.mean((gat - mu) ** 2, axis=0, keepdims=True)
    hh = g_ref[...] * (gat - mu) / jnp.sqrt(var + 1e-5) + be_ref[...]
    hh = jnp.maximum(hh, 0.0)
    h2 = jnp.dot(hh, w_ref[...], preferred_element_type=jnp.float32)
    h_ref[...] = h2
    hs_ref[...] = jnp.dot(h2, s_ref[...], preferred_element_type=jnp.float32)
    hd_ref[...] = jnp.dot(h2, d_ref[...], preferred_element_type=jnp.float32)

  return pl.pallas_call(
      body,
      out_shape=[
          jax.ShapeDtypeStruct((N, H2), jnp.float32),
          jax.ShapeDtypeStruct((N, 1), jnp.float32),
          jax.ShapeDtypeStruct((N, 1), jnp.float32),
      ],
  )(pr, b.reshape(1, H), g.reshape(1, H), be.reshape(1, H),
    W, a_src.reshape(H2, 1), a_dst.reshape(H2, 1))


def _tc_final(pr, b, g, be, Wd, bd):
  _, N, H = pr.shape
  EMB = Wd.shape[1]

  def body(pr_ref, b_ref, g_ref, be_ref, w_ref, bd_ref, o_ref):
    gat = pr_ref[0] + pr_ref[1] + b_ref[...]
    mu = jnp.mean(gat, axis=0, keepdims=True)
    var = jnp.mean((gat - mu) ** 2, axis=0, keepdims=True)
    hh = g_ref[...] * (gat - mu) / jnp.sqrt(var + 1e-5) + be_ref[...]
    hh = jnp.maximum(hh, 0.0)
    o_ref[...] = (jnp.dot(hh, w_ref[...], preferred_element_type=jnp.float32)
                  + bd_ref[...])

  return pl.pallas_call(
      body,
      out_shape=jax.ShapeDtypeStruct((N, EMB), jnp.float32),
  )(pr, b.reshape(1, H), g.reshape(1, H), be.reshape(1, H),
    Wd, bd.reshape(1, EMB))


@functools.partial(jax.jit, static_argnames=("N", "H", "E", "NG"))
def _sc_edge(hd2d, hs2d, src3, dst3, ridx, h, *, N, H, E, NG):
  NR = N // 16
  EC = NG * GW
  RPT = N // NS
  full, rem = divmod(RPT, GW)
  HC = H // 16

  mesh = plsc.VectorSubcoreMesh(core_axis_name="c", subcore_axis_name="s")

  @functools.partial(
      pl.kernel,
      out_type=jax.ShapeDtypeStruct((2, N, H), jnp.float32),
      mesh=mesh,
      scratch_types=[
          pltpu.VMEM((NR, 16), jnp.float32),
          pltpu.VMEM((NR, 16), jnp.float32),
          pltpu.VMEM((NG, GW), jnp.int32),
          pltpu.VMEM((NG, GW), jnp.int32),
          pltpu.VMEM((NG, GW), jnp.int32),
          pltpu.VMEM((NG, GW), jnp.int32),
          pltpu.VMEM((NR, 16), jnp.float32),
          pltpu.VMEM((NG, GW), jnp.float32),
          pltpu.VMEM((GW, H), jnp.float32),
          pltpu.VMEM((5, NR // 5), jnp.int32),
          pltpu.VMEM_SHARED((NR, 16), jnp.float32),
          pltpu.VMEM_SHARED((N, H), jnp.float32),
          pltpu.SemaphoreType.DMA,
      ],
  )
  def k(hd_hbm, hs_hbm, src_hbm, dst_hbm, ridx_hbm, h_hbm, out_hbm,
        hd_v, hs_v, srcO, dstO, srcM, dstM, sloc, exv, rowA, ridx_v,
        s_shared, out_shared, gsem):
    c = lax.axis_index("c")
    s = lax.axis_index("s")
    w_own = c * NS + s
    w_mir = (1 - c) * NS + s

    pltpu.sync_copy(hd_hbm, hd_v)
    pltpu.sync_copy(hs_hbm, hs_v)
    pltpu.sync_copy(src_hbm.at[w_own], srcO)
    pltpu.sync_copy(dst_hbm.at[w_own], dstO)
    pltpu.sync_copy(src_hbm.at[w_mir], srcM)
    pltpu.sync_copy(dst_hbm.at[w_mir], dstM)
    pltpu.sync_copy(ridx_hbm, ridx_v)

    def zero_s(i, _):
      sloc[i, :] = jnp.zeros((16,), jnp.float32)
      return 0
    lax.fori_loop(0, NR, zero_s, 0)

    def edge_scores(g, src_ref, dst_ref, base, store_ex):
      for kk in range(GW // 16):
        sl = pl.ds(kk * 16, 16)
        sv = src_ref[g, sl]
        dv = dst_ref[g, sl]
        ed = plsc.load_gather(hd_v, [dv >> 4, dv & 15])
        es = plsc.load_gather(hs_v, [sv >> 4, sv & 15])
        e = ed + es
        e = jnp.where(e >= 0, e, 0.2 * e)
        ex = jnp.exp(e)
        gidx = base + g * GW + kk * 16 + lax.iota(jnp.int32, 16)
        ex = jnp.where(gidx < E, ex, 0.0)
        if store_ex:
          exv[g, sl] = ex
        plsc.addupdate_scatter(sloc, [dv >> 4, dv & 15], ex)

    base_o = w_own * EC
    base_m = w_mir * EC

    def p1_own(g, _):
      edge_scores(g, srcO, dstO, base_o, True)
      return 0
    lax.fori_loop(0, NG, p1_own, 0)

    def p1_mir(g, _):
      edge_scores(g, srcM, dstM, base_m, False)
      return 0
    lax.fori_loop(0, NG, p1_mir, 0)

    plsc.subcore_barrier()

    @pl.when(s == 0)
    def _():
      pltpu.sync_copy(sloc, s_shared)
    plsc.subcore_barrier()

    @pl.when(s != 0)
    def _():
      for j in range(5):
        pltpu.sync_copy(sloc.at[pl.ds(j * (NR // 5), NR // 5)],
                        s_shared.at[ridx_v.at[j]], add=True)
    plsc.subcore_barrier()
    pltpu.sync_copy(s_shared, sloc)

    def alpha_loop(g, _):
      for kk in range(GW // 16):
        sl = pl.ds(kk * 16, 16)
        dv = dstO[g, sl]
        sv = plsc.load_gather(sloc, [dv >> 4, dv & 15])
        exv[g, sl] = exv[g, sl] / (sv + 1e-9)
      return 0
    lax.fori_loop(0, NG, alpha_loop, 0)

    def zrow(r, _):
      for kk in range(HC):
        rowA[r, pl.ds(kk * 16, 16)] = jnp.zeros((16,), jnp.float32)
      return 0
    lax.fori_loop(0, GW, zrow, 0)
    row0 = s * RPT
    for j in range(full):
      pltpu.sync_copy(rowA, out_shared.at[pl.ds(row0 + j * GW, GW)])
    if rem:
      pltpu.sync_copy(rowA.at[pl.ds(0, rem)],
                      out_shared.at[pl.ds(row0 + full * GW, rem)])
    plsc.subcore_barrier()

    def p2(g, _):
      pltpu.async_copy(h_hbm.at[srcO.at[g]], rowA, gsem).wait()

      def scale(e_i, _):
        a = exv[g, e_i]
        for kk in range(HC):
          sl = pl.ds(kk * 16, 16)
          rowA[e_i, sl] = rowA[e_i, sl] * a
        return 0
      lax.fori_loop(0, GW, scale, 0)
      pltpu.sync_copy(rowA, out_shared.at[dstO.at[g]], add=True)
      return 0
    lax.fori_loop(0, NG, p2, 0)
    plsc.subcore_barrier()

    for j in range(full):
      pltpu.sync_copy(out_shared.at[pl.ds(row0 + j * GW, GW)],
                      out_hbm.at[c, pl.ds(row0 + j * GW, GW)])
    if rem:
      pltpu.sync_copy(out_shared.at[pl.ds(row0 + full * GW, rem)],
                      out_hbm.at[c, pl.ds(row0 + full * GW, rem)])

  return k(hd2d, hs2d, src3, dst3, ridx, h)


def _sc_gather(full_rows, idx):
  N, EMB = full_rows.shape
  B = idx.shape[0]
  bw = B // NW

  mesh = plsc.VectorSubcoreMesh(core_axis_name="c", subcore_axis_name="s")

  @functools.partial(
      pl.kernel,
      out_type=jax.ShapeDtypeStruct((B, EMB), jnp.float32),
      mesh=mesh,
      scratch_types=[
          pltpu.VMEM((bw,), jnp.int32),
          pltpu.VMEM((bw, EMB), jnp.float32),
          pltpu.SemaphoreType.DMA,
      ],
  )
  def k(full_hbm, idx_hbm, out_hbm, idx_v, rows_v, sem):
    wid = lax.axis_index("s") * NC + lax.axis_index("c")
    base = wid * bw
    pltpu.sync_copy(idx_hbm.at[pl.ds(base, bw)], idx_v)
    pltpu.async_copy(full_hbm.at[idx_v], rows_v, sem).wait()
    pltpu.sync_copy(rows_v, out_hbm.at[pl.ds(base, bw)])

  return k(full_rows, idx)


def kernel(x, edge_index, idx, W1, a_src1, a_dst1, b1, g1, be1,
           W2, a_src2, a_dst2, b2, g2, be2, Wd, bd):
  N, _ = x.shape
  H = W1.shape[1]
  E = edge_index.shape[1]
  NR = N // 16

  NG = -(-E // (NW * GW))
  EP = NW * NG * GW
  pad = EP - E
  src = jnp.concatenate([edge_index[0], jnp.zeros((pad,), jnp.int32)])
  dst = jnp.concatenate([edge_index[1], jnp.zeros((pad,), jnp.int32)])
  src3 = src.reshape(NW, NG, GW)
  dst3 = dst.reshape(NW, NG, GW)
  ridx = jnp.arange(NR, dtype=jnp.int32).reshape(5, NR // 5)

  h1, hs1, hd1 = _tc_head(x, W1, a_src1, a_dst1)
  pr1 = _sc_edge(hd1.reshape(NR, 16), hs1.reshape(NR, 16),
                 src3, dst3, ridx, h1, N=N, H=H, E=E, NG=NG)
  h2, hs2, hd2 = _tc_mid(pr1, b1, g1, be1, W2, a_src2, a_dst2)
  pr2 = _sc_edge(hd2.reshape(NR, 16), hs2.reshape(NR, 16),
                 src3, dst3, ridx, h2, N=N, H=H, E=E, NG=NG)
  out_full = _tc_final(pr2, b2, g2, be2, Wd, bd)
  return _sc_gather(out_full, idx)


# speedup vs baseline: 17.3159x; 17.3159x over previous
import functools

import jax
import jax.numpy as jnp
from jax import lax
from jax.experimental import pallas as pl
from jax.experimental.pallas import tpu as pltpu
from jax.experimental.pallas import tpu_sc as plsc

NC = 2
NS = 16
NW = NC * NS
GW = 128


def _tc_head(x, W, a_src, a_dst):
  N, _ = x.shape
  H = W.shape[1]
  HH = H // 2

  QH = H // 4

  def body(x_ref, w_ref, s_ref, d_ref, h_ref, hs_ref, hd_ref):
    h = jnp.dot(x_ref[...], w_ref[...], preferred_element_type=jnp.float32)
    for q in range(4):
      h_ref[q * N:(q + 1) * N, :] = h[:, q * QH:(q + 1) * QH]
    hs_ref[...] = jnp.dot(h, s_ref[...], preferred_element_type=jnp.float32)
    hd_ref[...] = jnp.dot(h, d_ref[...], preferred_element_type=jnp.float32)

  return pl.pallas_call(
      body,
      compiler_params=pltpu.CompilerParams(vmem_limit_bytes=100 * 1024 * 1024),
      out_shape=[
          jax.ShapeDtypeStruct((4 * N, QH), jnp.float32),
          jax.ShapeDtypeStruct((N, 1), jnp.float32),
          jax.ShapeDtypeStruct((N, 1), jnp.float32),
      ],
  )(x, W, a_src.reshape(H, 1), a_dst.reshape(H, 1))


def _tc_mid(pr, b, g, be, W, a_src, a_dst):
  _, _, N, QH = pr.shape
  H = 4 * QH
  H2 = W.shape[1]
  QH2 = H2 // 4

  def body(pr_ref, b_ref, g_ref, be_ref, w_ref, s_ref, d_ref,
           h_ref, hs_ref, hd_ref):
    gat = jnp.concatenate(
        [pr_ref[0, 0], pr_ref[0, 1], pr_ref[1, 0], pr_ref[1, 1]],
        axis=-1) + b_ref[...]
    mu = jnp.mean(gat, axis=0, keepdims=True)
    var = jnp.mean((gat - mu) ** 2, axis=0, keepdims=True)
    hh = g_ref[...] * (gat - mu) / jnp.sqrt(var + 1e-5) + be_ref[...]
    hh = jnp.maximum(hh, 0.0)
    h2 = jnp.dot(hh, w_ref[...], preferred_element_type=jnp.float32)
    for q in range(4):
      h_ref[q * N:(q + 1) * N, :] = h2[:, q * QH2:(q + 1) * QH2]
    hs_ref[...] = jnp.dot(h2, s_ref[...], preferred_element_type=jnp.float32)
    hd_ref[...] = jnp.dot(h2, d_ref[...], preferred_element_type=jnp.float32)

  return pl.pallas_call(
      body,
      compiler_params=pltpu.CompilerParams(vmem_limit_bytes=100 * 1024 * 1024),
      out_shape=[
          jax.ShapeDtypeStruct((4 * N, QH2), jnp.float32),
          jax.ShapeDtypeStruct((N, 1), jnp.float32),
          jax.ShapeDtypeStruct((N, 1), jnp.float32),
      ],
  )(pr, b.reshape(1, H), g.reshape(1, H), be.reshape(1, H),
    W, a_src.reshape(H2, 1), a_dst.reshape(H2, 1))


def _tc_final(pr, b, g, be, Wd, bd):
  _, _, N, QH = pr.shape
  H = 4 * QH
  EMB = Wd.shape[1]

  def body(pr_ref, b_ref, g_ref, be_ref, w_ref, bd_ref, o_ref):
    gat = jnp.concatenate(
        [pr_ref[0, 0], pr_ref[0, 1], pr_ref[1, 0], pr_ref[1, 1]],
        axis=-1) + b_ref[...]
    mu = jnp.mean(gat, axis=0, keepdims=True)
    var = jnp.mean((gat - mu) ** 2, axis=0, keepdims=True)
    hh = g_ref[...] * (gat - mu) / jnp.sqrt(var + 1e-5) + be_ref[...]
    hh = jnp.maximum(hh, 0.0)
    o_ref[...] = (jnp.dot(hh, w_ref[...], preferred_element_type=jnp.float32)
                  + bd_ref[...])

  return pl.pallas_call(
      body,
      compiler_params=pltpu.CompilerParams(vmem_limit_bytes=100 * 1024 * 1024),
      out_shape=jax.ShapeDtypeStruct((N, EMB), jnp.float32),
  )(pr, b.reshape(1, H), g.reshape(1, H), be.reshape(1, H),
    Wd, bd.reshape(1, EMB))


@functools.partial(jax.jit, static_argnames=("N", "QH", "E", "NG"))
def _sc_edge(hd1, hs1, src3, dst3, h4n, *, N, QH, E, NG):
  EC = NG * GW
  SB = (N // NS // 8) * 8
  NEB = (N - NS * SB) // 8
  full, rem = divmod(SB, GW)
  HC = QH // 16
  SSEG = -(-N // NS // 16) * 16
  N2 = NS * SSEG

  mesh = plsc.VectorSubcoreMesh(core_axis_name="c", subcore_axis_name="s",
                                num_cores=NC, num_subcores=NS)

  @functools.partial(
      pl.kernel,
      out_type=jax.ShapeDtypeStruct((2, 2, N, QH), jnp.float32),
      mesh=mesh,
      scratch_types=[
          pltpu.VMEM((N,), jnp.float32),
          pltpu.VMEM((N,), jnp.float32),
          pltpu.VMEM((NG, GW), jnp.int32),
          pltpu.VMEM((NG, GW), jnp.int32),
          pltpu.VMEM((NG, GW), jnp.int32),
          pltpu.VMEM((NG, GW), jnp.int32),
          pltpu.VMEM((N2,), jnp.float32),
          pltpu.VMEM((NG, GW), jnp.float32),
          pltpu.VMEM((NG, GW), jnp.float32),
          pltpu.VMEM((GW, QH), jnp.float32),
          pltpu.VMEM((SSEG,), jnp.float32),
          pltpu.VMEM((SSEG,), jnp.float32),
          pltpu.VMEM_SHARED((NS * N2,), jnp.float32),
          pltpu.VMEM_SHARED((N2,), jnp.float32),
          pltpu.VMEM_SHARED((N, QH), jnp.float32),
          pltpu.SemaphoreType.DMA,
      ],
      compiler_params=pltpu.CompilerParams(needs_layout_passes=False,
                                           use_tc_tiling_on_sc=False),
  )
  def k(hd_hbm, hs_hbm, src_hbm, dst_hbm, h_hbm, out_hbm,
        hd_v, hs_v, srcA, dstA, srcB, dstB, sloc, exA, exB, rowA, sacc, stmp,
        s_all, s_shared, out_shared, gsem):
    c = lax.axis_index("c")
    s = lax.axis_index("s")
    w_a = c * NS + s
    w_b = (1 - c) * NS + s

    pltpu.sync_copy(hd_hbm, hd_v)
    pltpu.sync_copy(hs_hbm, hs_v)
    pltpu.sync_copy(src_hbm.at[w_a], srcA)
    pltpu.sync_copy(dst_hbm.at[w_a], dstA)
    pltpu.sync_copy(src_hbm.at[w_b], srcB)
    pltpu.sync_copy(dst_hbm.at[w_b], dstB)

    def zero_s(i, _):
      sloc[pl.ds(pl.multiple_of(i * 16, 16), 16)] = jnp.zeros((16,),
                                                              jnp.float32)
      return 0
    lax.fori_loop(0, N2 // 16, zero_s, 0)

    def edge_scores(g, src_ref, dst_ref, ex_ref, base):
      for kk in range(GW // 16):
        sl = pl.ds(kk * 16, 16)
        sv = src_ref[g, sl]
        dv = dst_ref[g, sl]
        ed = plsc.load_gather(hd_v, [dv])
        es = plsc.load_gather(hs_v, [sv])
        e = ed + es
        e = jnp.where(e >= 0, e, 0.2 * e)
        ex = jnp.exp(e)
        gidx = base + g * GW + kk * 16 + lax.iota(jnp.int32, 16)
        ex = jnp.where(gidx < E, ex, 0.0)
        ex_ref[g, sl] = ex
        plsc.addupdate_scatter(sloc, [dv], ex)

    base_a = w_a * EC
    base_b = w_b * EC

    def p1_a(g, _):
      edge_scores(g, srcA, dstA, exA, base_a)
      return 0
    lax.fori_loop(0, NG, p1_a, 0)

    def p1_b(g, _):
      edge_scores(g, srcB, dstB, exB, base_b)
      return 0
    lax.fori_loop(0, NG, p1_b, 0)

    plsc.subcore_barrier()
    sbase = pl.multiple_of(s * N2, 8)
    pltpu.sync_copy(sloc, s_all.at[pl.ds(sbase, N2)])
    plsc.subcore_barrier()
    seg0 = pl.multiple_of(s * SSEG, 8)
    pltpu.sync_copy(s_all.at[pl.ds(seg0, SSEG)], sacc)
    for t in range(1, NS):
      pltpu.sync_copy(
          s_all.at[pl.ds(pl.multiple_of(t * N2, 8) + seg0, SSEG)], stmp)

      def accum(q, _):
        sl = pl.ds(pl.multiple_of(q * 16, 16), 16)
        sacc[sl] = sacc[sl] + stmp[sl]
        return 0
      lax.fori_loop(0, SSEG // 16, accum, 0)
    pltpu.sync_copy(sacc, s_shared.at[pl.ds(seg0, SSEG)])
    plsc.subcore_barrier()
    pltpu.sync_copy(s_shared, sloc)

    coff = c * (2 * N)

    def alpha_loop(g, _):
      for (dst_ref, src_ref, ex_ref) in ((dstA, srcA, exA),
                                         (dstB, srcB, exB)):
        for kk in range(GW // 16):
          sl = pl.ds(kk * 16, 16)
          dv = dst_ref[g, sl]
          sv = plsc.load_gather(sloc, [dv])
          ex_ref[g, sl] = ex_ref[g, sl] / (sv + 1e-9)
          src_ref[g, sl] = src_ref[g, sl] + coff
      return 0
    lax.fori_loop(0, NG, alpha_loop, 0)

    row0 = pl.multiple_of(s * SB, 8)
    xrow = pl.multiple_of(NS * SB + s * 8, 8)

    def p2_body(g, src_ref, dst_ref, ex_ref):
      pltpu.async_copy(h_hbm.at[src_ref.at[g]], rowA, gsem).wait()

      def scale(q, _):
        a16 = ex_ref[g, pl.ds(q * 16, 16)]
        for j in range(16):
          a = a16[j]
          r = q * 16 + j
          for kk in range(HC):
            sl = pl.ds(kk * 16, 16)
            rowA[r, sl] = rowA[r, sl] * a
        return 0
      lax.fori_loop(0, GW // 16, scale, 0)
      pltpu.sync_copy(rowA, out_shared.at[dst_ref.at[g]], add=True)

    for ep in range(2):
      def zrow(r, _):
        for kk in range(HC):
          rowA[r, pl.ds(kk * 16, 16)] = jnp.zeros((16,), jnp.float32)
        return 0
      lax.fori_loop(0, GW, zrow, 0)
      for j in range(full):
        pltpu.sync_copy(rowA, out_shared.at[pl.ds(row0 + j * GW, GW)])
      if rem:
        pltpu.sync_copy(rowA.at[pl.ds(0, rem)],
                        out_shared.at[pl.ds(row0 + full * GW, rem)])

      @pl.when(s < NEB)
      def _():
        pltpu.sync_copy(rowA.at[pl.ds(0, 8)], out_shared.at[pl.ds(xrow, 8)])
      plsc.subcore_barrier()

      def p2_a(g, _):
        p2_body(g, srcA, dstA, exA)
        return 0
      lax.fori_loop(0, NG, p2_a, 0)

      def p2_b(g, _):
        p2_body(g, srcB, dstB, exB)
        return 0
      lax.fori_loop(0, NG, p2_b, 0)
      plsc.subcore_barrier()

      pltpu.sync_copy(out_shared.at[pl.ds(row0, SB)],
                      out_hbm.at[c, ep, pl.ds(row0, SB)])

      @pl.when(s < NEB)
      def _():
        pltpu.sync_copy(out_shared.at[pl.ds(xrow, 8)],
                        out_hbm.at[c, ep, pl.ds(xrow, 8)])

      if ep == 0:
        def bump(g, _):
          for src_ref in (srcA, srcB):
            for kk in range(GW // 16):
              sl = pl.ds(kk * 16, 16)
              src_ref[g, sl] = src_ref[g, sl] + N
          return 0
        lax.fori_loop(0, NG, bump, 0)

  return k(hd1, hs1, src3, dst3, h4n)


def _sc_gather(full_rows, idx):
  N, EMB = full_rows.shape
  B = idx.shape[0]
  bw = B // NW

  mesh = plsc.VectorSubcoreMesh(core_axis_name="c", subcore_axis_name="s",
                                num_cores=NC, num_subcores=NS)

  @functools.partial(
      pl.kernel,
      out_type=jax.ShapeDtypeStruct((B, EMB), jnp.float32),
      mesh=mesh,
      scratch_types=[
          pltpu.VMEM((bw,), jnp.int32),
          pltpu.VMEM((bw, EMB), jnp.float32),
          pltpu.SemaphoreType.DMA,
      ],
  )
  def k(full_hbm, idx_hbm, out_hbm, idx_v, rows_v, sem):
    wid = lax.axis_index("s") * NC + lax.axis_index("c")
    base = wid * bw
    pltpu.sync_copy(idx_hbm.at[pl.ds(base, bw)], idx_v)
    pltpu.async_copy(full_hbm.at[idx_v], rows_v, sem).wait()
    pltpu.sync_copy(rows_v, out_hbm.at[pl.ds(base, bw)])

  return k(full_rows, idx)


def kernel(x, edge_index, idx, W1, a_src1, a_dst1, b1, g1, be1,
           W2, a_src2, a_dst2, b2, g2, be2, Wd, bd):
  N, _ = x.shape
  H = W1.shape[1]
  E = edge_index.shape[1]

  NG = -(-E // (NW * GW))
  EP = NW * NG * GW
  pad = EP - E
  src = jnp.concatenate([edge_index[0], jnp.zeros((pad,), jnp.int32)])
  dst = jnp.concatenate([edge_index[1], jnp.zeros((pad,), jnp.int32)])
  src3 = src.reshape(NW, NG, GW)
  dst3 = dst.reshape(NW, NG, GW)

  h1, hs1, hd1 = _tc_head(x, W1, a_src1, a_dst1)
  pr1 = _sc_edge(hd1.reshape(N), hs1.reshape(N),
                 src3, dst3, h1, N=N, QH=H // 4, E=E, NG=NG)
  h2, hs2, hd2 = _tc_mid(pr1, b1, g1, be1, W2, a_src2, a_dst2)
  pr2 = _sc_edge(hd2.reshape(N), hs2.reshape(N),
                 src3, dst3, h2, N=N, QH=W2.shape[1] // 4, E=E, NG=NG)
  out_full = _tc_final(pr2, b2, g2, be2, Wd, bd)
  return _sc_gather(out_full, idx)
